# Initial kernel scaffold; baseline (speedup 1.0000x reference)
import functools

import jax
import jax.numpy as jnp
import numpy as np
from jax import lax
from jax.experimental import pallas as pl
from jax.experimental.pallas import tpu as pltpu
from jax.experimental.pallas import tpu_sc as plsc

_D = 10
_V = 27
_CHUNK = 4096


@functools.cache
def _make_kernel(n_flat):
  info = plsc.get_sparse_core_info()
  nc, ns = info.num_cores, info.num_subcores
  nw = nc * ns
  per_w = n_flat // nw
  n_chunks = per_w // _CHUNK
  groups = _CHUNK // 16
  mesh = plsc.VectorSubcoreMesh(core_axis_name="c", subcore_axis_name="s")

  ej = [np.arange(16 * j, 16 * j + 16, dtype=np.int32) // _D for j in range(_D)]
  dj = [np.arange(16 * j, 16 * j + 16, dtype=np.int32) % _D for j in range(_D)]

  @functools.partial(
      pl.kernel,
      out_type=jax.ShapeDtypeStruct((n_flat * _D,), jnp.float32),
      mesh=mesh,
      scratch_types=[
          pltpu.VMEM((_V, _D), jnp.float32),
          pltpu.VMEM((_CHUNK,), jnp.int32),
          pltpu.VMEM((_CHUNK * _D,), jnp.float32),
      ],
  )
  def k(x_hbm, w_hbm, out_hbm, table, xv, outv):
    wid = lax.axis_index("s") * nc + lax.axis_index("c")
    pltpu.sync_copy(w_hbm, table)
    ejc = [jnp.asarray(e) for e in ej]
    djc = [jnp.asarray(d) for d in dj]

    def chunk_body(c, carry):
      base = wid * per_w + c * _CHUNK
      pltpu.sync_copy(x_hbm.at[pl.ds(base, _CHUNK)], xv)

      def group_body(t, carry):
        x16 = xv[pl.ds(t * 16, 16)]
        for j in range(_D):
          xe = jnp.take(x16, ejc[j], mode="promise_in_bounds")
          vals = plsc.load_gather(table, [xe, djc[j]])
          outv[pl.ds(t * 160 + j * 16, 16)] = vals
        return carry

      lax.fori_loop(0, groups, group_body, 0)
      pltpu.sync_copy(outv, out_hbm.at[pl.ds(base * _D, _CHUNK * _D)])
      return carry

    lax.fori_loop(0, n_chunks, chunk_body, 0)

  return k


@jax.jit
def kernel(x, weight):
  n_flat = x.shape[0] * x.shape[1]
  out = _make_kernel(n_flat)(x.reshape(n_flat), weight)
  return out.reshape(x.shape[0], x.shape[1], _D)


# speedup vs baseline: 4.3752x; 4.3752x over previous
import functools

import jax
import jax.numpy as jnp
import numpy as np
from jax import lax
from jax.experimental import pallas as pl
from jax.experimental.pallas import tpu as pltpu
from jax.experimental.pallas import tpu_sc as plsc

_D = 10
_V = 27
_TFLAT = 280
_CHUNK = 4096


@functools.cache
def _make_kernel(n_flat):
  info = plsc.get_sparse_core_info()
  nc, ns = info.num_cores, info.num_subcores
  nw = nc * ns
  per_w = n_flat // nw
  n_chunks = per_w // _CHUNK
  groups = _CHUNK // 16
  mesh = plsc.VectorSubcoreMesh(core_axis_name="c", subcore_axis_name="s")

  @functools.partial(
      pl.kernel,
      out_type=jax.ShapeDtypeStruct((n_flat * _D,), jnp.float32),
      mesh=mesh,
      compiler_params=pltpu.CompilerParams(needs_layout_passes=False),
      scratch_types=[
          pltpu.VMEM((_TFLAT,), jnp.float32),
          pltpu.VMEM((_D, 16), jnp.int32),
          pltpu.VMEM((_D, 16), jnp.int32),
          pltpu.VMEM((_CHUNK,), jnp.int32),
          pltpu.VMEM((_CHUNK * _D,), jnp.float32),
      ],
  )
  def k(x_hbm, w_hbm, emap_hbm, dmap_hbm, out_hbm, table, emap, dmap, xv, outv):
    wid = lax.axis_index("s") * nc + lax.axis_index("c")
    pltpu.sync_copy(w_hbm, table)
    pltpu.sync_copy(emap_hbm, emap)
    pltpu.sync_copy(dmap_hbm, dmap)
    ejc = [emap[j] for j in range(_D)]
    djc = [dmap[j] for j in range(_D)]

    def chunk_body(c, carry):
      base = wid * per_w + c * _CHUNK
      pltpu.sync_copy(x_hbm.at[pl.ds(base, _CHUNK)], xv)

      def group_body(t, carry):
        x16 = xv[pl.ds(t * 16, 16)] * _D
        for j in range(_D):
          xe = jnp.take_along_axis(x16, ejc[j], axis=0, mode="promise_in_bounds")
          vals = plsc.load_gather(table, [xe + djc[j]])
          outv[pl.ds(t * 160 + j * 16, 16)] = vals
        return carry

      lax.fori_loop(0, groups, group_body, 0)
      pltpu.sync_copy(outv, out_hbm.at[pl.ds(base * _D, _CHUNK * _D)])
      return carry

    lax.fori_loop(0, n_chunks, chunk_body, 0)

  return k


_POS = np.arange(160, dtype=np.int32).reshape(_D, 16)
_EMAP = _POS // _D
_DMAP = _POS % _D


@jax.jit
def kernel(x, weight):
  n_flat = x.shape[0] * x.shape[1]
  wflat = jnp.zeros((_TFLAT,), jnp.float32).at[: _V * _D].set(weight.reshape(-1))
  out = _make_kernel(n_flat)(
      x.reshape(n_flat),
      wflat,
      jnp.asarray(_EMAP),
      jnp.asarray(_DMAP),
  )
  return out.reshape(x.shape[0], x.shape[1], _D)
